# Initial kernel scaffold; baseline (speedup 1.0000x reference)
import jax
import jax.numpy as jnp
from jax.experimental import pallas as pl


def kernel(pred, target):
    raise NotImplementedError("write your pallas kernel here")


# speedup vs baseline: 16.0519x; 16.0519x over previous
import functools

import jax
import jax.numpy as jnp
from jax.experimental import pallas as pl
from jax.experimental.pallas import tpu as pltpu

_OHEM_RATIO = 0.7
_EPS = 1e-07


def _fused_kernel(pred_ref, tgt_ref, out_ref, bits_ref, mm_ref, *, k, n_steps):
    i = pl.program_id(0) * pl.num_programs(1) + pl.program_id(1)

    x = pred_ref[0]
    t = tgt_ref[0]
    m = jnp.max(x, axis=0)
    e = jnp.exp(x - m[None, :, :])
    s = jnp.sum(e, axis=0)
    lse = jnp.log(s) + m
    cls = jax.lax.broadcasted_iota(jnp.int32, x.shape, 0)
    tl = jnp.sum(jnp.where(cls == t[None, :, :], x, 0.0), axis=0)
    loss = jnp.maximum(lse - tl, 0.0)
    blk = jax.lax.bitcast_convert_type(loss, jnp.int32)
    bits_ref[i] = blk

    @pl.when(i == 0)
    def _init():
        mm_ref[0] = jnp.int32(0x7F800000)
        mm_ref[1] = jnp.int32(0)

    mm_ref[0] = jnp.minimum(mm_ref[0], jnp.min(blk))
    mm_ref[1] = jnp.maximum(mm_ref[1], jnp.max(blk))

    @pl.when(i == n_steps - 1)
    def _select():
        th, w = bits_ref.shape[1], bits_ref.shape[2]
        zero = jnp.zeros((th, w), jnp.int32)

        def count3(t1, t2, t3):
            def chunk(c, accs):
                a1, a2, a3 = accs
                ch = bits_ref[c]
                one = jnp.int32(1)
                a1 = a1 + jnp.where(ch >= t1, one, 0)
                a2 = a2 + jnp.where(ch >= t2, one, 0)
                a3 = a3 + jnp.where(ch >= t3, one, 0)
                return a1, a2, a3

            a1, a2, a3 = jax.lax.fori_loop(0, n_steps, chunk,
                                           (zero, zero, zero))
            return jnp.sum(a1), jnp.sum(a2), jnp.sum(a3)

        def body(carry):
            lo, hi = carry
            q = (hi - lo) // 4 + 1
            t1, t2, t3 = lo + q, lo + 2 * q, lo + 3 * q
            c1, c2, c3 = count3(t1, t2, t3)
            new_lo = jnp.where(c3 >= k, t3,
                               jnp.where(c2 >= k, t2,
                                         jnp.where(c1 >= k, t1, lo)))
            new_hi = jnp.where(c1 < k, t1 - 1,
                               jnp.where(c2 < k, t2 - 1,
                                         jnp.where(c3 < k, t3 - 1, hi)))
            return new_lo, new_hi

        lo, _ = jax.lax.while_loop(lambda c: c[0] < c[1], body,
                                   (mm_ref[0], mm_ref[1]))

        def final_chunk(c, accs):
            asum, acnt = accs
            chb = bits_ref[c]
            chv = jax.lax.bitcast_convert_type(chb, jnp.float32)
            sel = chb >= lo
            asum = asum + jnp.where(sel, chv, 0.0)
            acnt = acnt + jnp.where(sel, 1, 0)
            return asum, acnt

        asum, acnt = jax.lax.fori_loop(
            0, n_steps, final_chunk,
            (jnp.zeros((th, w), jnp.float32), zero))
        total = jnp.sum(asum)
        cnt = jnp.sum(acnt).astype(jnp.float32)
        out_ref[...] = jnp.reshape(total / (cnt + _EPS), (1, 1))


def kernel(pred, target):
    B, C, H, W = pred.shape
    TH = 64
    N = B * H * W
    k = int(N * _OHEM_RATIO)
    n_h = H // TH
    n_steps = B * n_h
    out = pl.pallas_call(
        functools.partial(_fused_kernel, k=k, n_steps=n_steps),
        grid=(B, n_h),
        in_specs=[
            pl.BlockSpec((1, C, TH, W), lambda b, h: (b, 0, h, 0)),
            pl.BlockSpec((1, TH, W), lambda b, h: (b, h, 0)),
        ],
        out_specs=pl.BlockSpec((1, 1), lambda b, h: (0, 0)),
        out_shape=jax.ShapeDtypeStruct((1, 1), jnp.float32),
        scratch_shapes=[pltpu.VMEM((n_steps, TH, W), jnp.int32),
                        pltpu.SMEM((2,), jnp.int32)],
    )(pred, target)
    return out[0, 0]


# speedup vs baseline: 17.7437x; 1.1054x over previous
import functools

import jax
import jax.numpy as jnp
from jax.experimental import pallas as pl
from jax.experimental.pallas import tpu as pltpu

_OHEM_RATIO = 0.7
_EPS = 1e-07


def _fused_kernel(pred_ref, tgt_ref, out_ref, bits_ref, mm_ref, *, k, n_steps):
    i = pl.program_id(0) * pl.num_programs(1) + pl.program_id(1)

    x = pred_ref[0]
    t = tgt_ref[0]
    m = jnp.max(x, axis=0)
    e = jnp.exp(x - m[None, :, :])
    s = jnp.sum(e, axis=0)
    lse = jnp.log(s) + m
    cls = jax.lax.broadcasted_iota(jnp.int32, x.shape, 0)
    tl = jnp.sum(jnp.where(cls == t[None, :, :], x, 0.0), axis=0)
    loss = jnp.maximum(lse - tl, 0.0)
    blk = jax.lax.bitcast_convert_type(loss, jnp.int32)
    bits_ref[i] = blk

    @pl.when(i == 0)
    def _init():
        mm_ref[0] = jnp.int32(0x7F800000)
        mm_ref[1] = jnp.int32(0)

    mm_ref[0] = jnp.minimum(mm_ref[0], jnp.min(blk))
    mm_ref[1] = jnp.maximum(mm_ref[1], jnp.max(blk))

    @pl.when(i == n_steps - 1)
    def _select():
        bits = bits_ref[...]

        def count(thr):
            part = jnp.sum((bits >= thr).astype(jnp.int32), axis=0)
            return jnp.sum(part)

        def body(carry):
            lo, hi = carry
            mid = lo + (hi - lo + 1) // 2
            ok = count(mid) >= k
            return jnp.where(ok, mid, lo), jnp.where(ok, hi, mid - 1)

        lo, _ = jax.lax.while_loop(lambda c: c[0] < c[1], body,
                                   (mm_ref[0], mm_ref[1]))

        sel = bits >= lo
        v = jax.lax.bitcast_convert_type(bits, jnp.float32)
        total = jnp.sum(jnp.sum(jnp.where(sel, v, 0.0), axis=0))
        cnt = jnp.sum(jnp.sum(sel.astype(jnp.int32), axis=0))
        out_ref[...] = jnp.reshape(total / (cnt.astype(jnp.float32) + _EPS),
                                   (1, 1))


def kernel(pred, target):
    B, C, H, W = pred.shape
    TH = 64
    N = B * H * W
    k = int(N * _OHEM_RATIO)
    n_h = H // TH
    n_steps = B * n_h
    out = pl.pallas_call(
        functools.partial(_fused_kernel, k=k, n_steps=n_steps),
        grid=(B, n_h),
        in_specs=[
            pl.BlockSpec((1, C, TH, W), lambda b, h: (b, 0, h, 0)),
            pl.BlockSpec((1, TH, W), lambda b, h: (b, h, 0)),
        ],
        out_specs=pl.BlockSpec((1, 1), lambda b, h: (0, 0)),
        out_shape=jax.ShapeDtypeStruct((1, 1), jnp.float32),
        scratch_shapes=[pltpu.VMEM((n_steps, TH, W), jnp.int32),
                        pltpu.SMEM((2,), jnp.int32)],
    )(pred, target)
    return out[0, 0]


# speedup vs baseline: 23.9110x; 1.3476x over previous
import functools

import jax
import jax.numpy as jnp
from jax.experimental import pallas as pl
from jax.experimental.pallas import tpu as pltpu

_OHEM_RATIO = 0.7
_EPS = 1e-07
_NR = 16
_STEP = 16384


def _fused_kernel(pred_ref, tgt_ref, out_ref, bits_ref, mm_ref, acc_ref,
                  *, k, n_steps):
    i = pl.program_id(0) * pl.num_programs(1) + pl.program_id(1)

    x = pred_ref[0]
    t = tgt_ref[0]
    m = jnp.max(x, axis=0)
    e = jnp.exp(x - m[None, :, :])
    s = jnp.sum(e, axis=0)
    lse = jnp.log(s) + m
    cls = jax.lax.broadcasted_iota(jnp.int32, x.shape, 0)
    tl = jnp.sum(jnp.where(cls == t[None, :, :], x, 0.0), axis=0)
    loss = jnp.maximum(lse - tl, 0.0)
    blk = jax.lax.bitcast_convert_type(loss, jnp.int32)
    bits_ref[i] = blk

    @pl.when(i == 0)
    def _init():
        mm_ref[0] = jnp.int32(0x7F800000)
        mm_ref[1] = jnp.int32(0)
        kb = k // n_steps

        def pbody(_, carry):
            lo, hi = carry
            mid = lo + (hi - lo + 1) // 2
            cnt = jnp.sum(jnp.sum((blk >= mid).astype(jnp.int32), axis=0))
            ok = cnt >= kb
            return jnp.where(ok, mid, lo), jnp.where(ok, hi, mid - 1)

        plo, _ = jax.lax.fori_loop(0, 31, pbody,
                                   (jnp.int32(0), jnp.int32(0x7F800000)))
        mm_ref[2] = plo
        acc_ref[...] = jnp.zeros_like(acc_ref)

    mm_ref[0] = jnp.minimum(mm_ref[0], jnp.min(blk))
    mm_ref[1] = jnp.maximum(mm_ref[1], jnp.max(blk))

    p = mm_ref[2]
    for j in range(_NR):
        tj = p + jnp.int32((j - _NR // 2) * _STEP)
        acc_ref[j] = acc_ref[j] + jnp.where(blk >= tj, 1, 0)

    @pl.when(i == n_steps - 1)
    def _select():
        bits = bits_ref[...]

        def count(thr):
            part = jnp.sum((bits >= thr).astype(jnp.int32), axis=0)
            return jnp.sum(part)

        def rung(j, carry):
            lo, hi = carry
            tj = p + (j - _NR // 2) * _STEP
            cj = jnp.sum(jnp.sum(acc_ref[j], axis=0))
            lo = jnp.where(cj >= k, jnp.maximum(lo, tj), lo)
            hi = jnp.where(cj < k, jnp.minimum(hi, tj - 1), hi)
            return lo, hi

        lo0, hi0 = jax.lax.fori_loop(0, _NR, rung, (mm_ref[0], mm_ref[1]))

        def body(carry):
            lo, hi = carry
            q = (hi - lo) // 4 + 1
            t1, t2, t3 = lo + q, lo + 2 * q, lo + 3 * q
            c1, c2, c3 = count(t1), count(t2), count(t3)
            new_lo = jnp.where(c3 >= k, t3,
                               jnp.where(c2 >= k, t2,
                                         jnp.where(c1 >= k, t1, lo)))
            new_hi = jnp.where(c1 < k, t1 - 1,
                               jnp.where(c2 < k, t2 - 1,
                                         jnp.where(c3 < k, t3 - 1, hi)))
            return new_lo, new_hi

        lo, _ = jax.lax.while_loop(lambda c: c[0] < c[1], body, (lo0, hi0))

        sel = bits >= lo
        v = jax.lax.bitcast_convert_type(bits, jnp.float32)
        total = jnp.sum(jnp.sum(jnp.where(sel, v, 0.0), axis=0))
        cnt = jnp.sum(jnp.sum(sel.astype(jnp.int32), axis=0))
        out_ref[...] = jnp.reshape(total / (cnt.astype(jnp.float32) + _EPS),
                                   (1, 1))


def kernel(pred, target):
    B, C, H, W = pred.shape
    TH = 64
    N = B * H * W
    k = int(N * _OHEM_RATIO)
    n_h = H // TH
    n_steps = B * n_h
    out = pl.pallas_call(
        functools.partial(_fused_kernel, k=k, n_steps=n_steps),
        grid=(B, n_h),
        in_specs=[
            pl.BlockSpec((1, C, TH, W), lambda b, h: (b, 0, h, 0)),
            pl.BlockSpec((1, TH, W), lambda b, h: (b, h, 0)),
        ],
        out_specs=pl.BlockSpec((1, 1), lambda b, h: (0, 0)),
        out_shape=jax.ShapeDtypeStruct((1, 1), jnp.float32),
        scratch_shapes=[pltpu.VMEM((n_steps, TH, W), jnp.int32),
                        pltpu.SMEM((3,), jnp.int32),
                        pltpu.VMEM((_NR, TH, W), jnp.int32)],
    )(pred, target)
    return out[0, 0]


# speedup vs baseline: 27.7563x; 1.1608x over previous
import functools

import jax
import jax.numpy as jnp
from jax.experimental import pallas as pl
from jax.experimental.pallas import tpu as pltpu

_OHEM_RATIO = 0.7
_EPS = 1e-07
_NR = 16
_STEP = 16384


def _fused_kernel(pred_ref, tgt_ref, out_ref, bits_ref, mm_ref, acc_ref,
                  *, k, n_steps):
    i = pl.program_id(0) * pl.num_programs(1) + pl.program_id(1)

    x = pred_ref[0]
    t = tgt_ref[0]
    m = jnp.max(x, axis=0)
    e = jnp.exp(x - m[None, :, :])
    s = jnp.sum(e, axis=0)
    lse = jnp.log(s) + m
    cls = jax.lax.broadcasted_iota(jnp.int32, x.shape, 0)
    tl = jnp.sum(jnp.where(cls == t[None, :, :], x, 0.0), axis=0)
    loss = jnp.maximum(lse - tl, 0.0)
    blk = jax.lax.bitcast_convert_type(loss, jnp.int32)
    bits_ref[i] = blk

    @pl.when(i == 0)
    def _init():
        mm_ref[0] = jnp.int32(0x7F800000)
        mm_ref[1] = jnp.int32(0)
        kb = k // n_steps

        def pbody(_, carry):
            lo, hi = carry
            mid = lo + (hi - lo + 1) // 2
            cnt = jnp.sum(jnp.sum((blk >= mid).astype(jnp.int32), axis=0))
            ok = cnt >= kb
            return jnp.where(ok, mid, lo), jnp.where(ok, hi, mid - 1)

        plo, _ = jax.lax.fori_loop(0, 31, pbody,
                                   (jnp.int32(0), jnp.int32(0x7F800000)))
        mm_ref[2] = plo
        acc_ref[...] = jnp.zeros_like(acc_ref)

    mm_ref[0] = jnp.minimum(mm_ref[0], jnp.min(blk))
    mm_ref[1] = jnp.maximum(mm_ref[1], jnp.max(blk))

    p = mm_ref[2]
    for j in range(_NR):
        tj = p + jnp.int32((j - _NR // 2) * _STEP)
        acc_ref[j] = acc_ref[j] + jnp.where(blk >= tj, 1, 0)

    @pl.when(i == n_steps - 1)
    def _select():
        bits = bits_ref[...]

        def count(thr):
            part = jnp.sum((bits >= thr).astype(jnp.int32), axis=0)
            return jnp.sum(part)

        def rung(j, carry):
            lo, hi = carry
            tj = p + (j - _NR // 2) * _STEP
            cj = jnp.sum(jnp.sum(acc_ref[j], axis=0))
            lo = jnp.where(cj >= k, jnp.maximum(lo, tj), lo)
            hi = jnp.where(cj < k, jnp.minimum(hi, tj - 1), hi)
            return lo, hi

        lo0, hi0 = jax.lax.fori_loop(0, _NR, rung, (mm_ref[0], mm_ref[1]))

        def body(carry):
            lo, hi = carry
            q = (hi - lo) // 4 + 1
            t1, t2, t3 = lo + q, lo + 2 * q, lo + 3 * q
            c1, c2, c3 = count(t1), count(t2), count(t3)
            new_lo = jnp.where(c3 >= k, t3,
                               jnp.where(c2 >= k, t2,
                                         jnp.where(c1 >= k, t1, lo)))
            new_hi = jnp.where(c1 < k, t1 - 1,
                               jnp.where(c2 < k, t2 - 1,
                                         jnp.where(c3 < k, t3 - 1, hi)))
            return new_lo, new_hi

        lo, _ = jax.lax.while_loop(lambda c: c[0] < c[1], body, (lo0, hi0))

        sel = bits >= lo
        v = jax.lax.bitcast_convert_type(bits, jnp.float32)
        total = jnp.sum(jnp.sum(jnp.where(sel, v, 0.0), axis=0))
        cnt = jnp.sum(jnp.sum(sel.astype(jnp.int32), axis=0))
        out_ref[...] = jnp.reshape(total / (cnt.astype(jnp.float32) + _EPS),
                                   (1, 1))


def kernel(pred, target):
    B, C, H, W = pred.shape
    TH = min(128, H)
    N = B * H * W
    k = int(N * _OHEM_RATIO)
    n_h = H // TH
    n_steps = B * n_h
    out = pl.pallas_call(
        functools.partial(_fused_kernel, k=k, n_steps=n_steps),
        grid=(B, n_h),
        in_specs=[
            pl.BlockSpec((1, C, TH, W), lambda b, h: (b, 0, h, 0)),
            pl.BlockSpec((1, TH, W), lambda b, h: (b, h, 0)),
        ],
        out_specs=pl.BlockSpec((1, 1), lambda b, h: (0, 0)),
        out_shape=jax.ShapeDtypeStruct((1, 1), jnp.float32),
        scratch_shapes=[pltpu.VMEM((n_steps, TH, W), jnp.int32),
                        pltpu.SMEM((3,), jnp.int32),
                        pltpu.VMEM((_NR, TH, W), jnp.int32)],
    )(pred, target)
    return out[0, 0]


# speedup vs baseline: 28.0453x; 1.0104x over previous
import functools

import jax
import jax.numpy as jnp
from jax.experimental import pallas as pl
from jax.experimental.pallas import tpu as pltpu

_OHEM_RATIO = 0.7
_EPS = 1e-07
_NR = 16
_STEP = 16384


def _fused_kernel(pred_ref, tgt_ref, out_ref, bits_ref, mm_ref, acc_ref,
                  *, k, n_steps):
    i = pl.program_id(0) * pl.num_programs(1) + pl.program_id(1)

    x = pred_ref[0]
    t = tgt_ref[0]
    m = jnp.max(x, axis=0)
    e = jnp.exp(x - m[None, :, :])
    s = jnp.sum(e, axis=0)
    lse = jnp.log(s) + m
    cls = jax.lax.broadcasted_iota(jnp.int32, x.shape, 0)
    tl = jnp.sum(jnp.where(cls == t[None, :, :], x, 0.0), axis=0)
    loss = jnp.maximum(lse - tl, 0.0)
    blk = jax.lax.bitcast_convert_type(loss, jnp.int32)
    bits_ref[i] = blk

    @pl.when(i == 0)
    def _init():
        mm_ref[0] = jnp.int32(0x7F800000)
        mm_ref[1] = jnp.int32(0)
        kb = k // n_steps

        def pbody(_, carry):
            lo, hi = carry
            mid = lo + (hi - lo + 1) // 2
            cnt = jnp.sum(jnp.sum((blk >= mid).astype(jnp.int32), axis=0))
            ok = cnt >= kb
            return jnp.where(ok, mid, lo), jnp.where(ok, hi, mid - 1)

        plo, _ = jax.lax.fori_loop(0, 18, pbody,
                                   (jnp.int32(0), jnp.int32(0x7F800000)))
        mm_ref[2] = plo
        acc_ref[...] = jnp.zeros_like(acc_ref)

    mm_ref[0] = jnp.minimum(mm_ref[0], jnp.min(blk))
    mm_ref[1] = jnp.maximum(mm_ref[1], jnp.max(blk))

    p = mm_ref[2]
    for j in range(_NR):
        tj = p + jnp.int32((j - _NR // 2) * _STEP)
        acc_ref[j] = acc_ref[j] + jnp.where(blk >= tj, 1, 0)

    @pl.when(i == n_steps - 1)
    def _select():
        bits = bits_ref[...]

        def count(thr):
            part = jnp.sum((bits >= thr).astype(jnp.int32), axis=0)
            return jnp.sum(part)

        def rung(j, carry):
            lo, hi = carry
            tj = p + (j - _NR // 2) * _STEP
            cj = jnp.sum(jnp.sum(acc_ref[j], axis=0))
            lo = jnp.where(cj >= k, jnp.maximum(lo, tj), lo)
            hi = jnp.where(cj < k, jnp.minimum(hi, tj - 1), hi)
            return lo, hi

        lo0, hi0 = jax.lax.fori_loop(0, _NR, rung, (mm_ref[0], mm_ref[1]))

        def body(carry):
            lo, hi = carry
            q = (hi - lo) // 4 + 1
            t1, t2, t3 = lo + q, lo + 2 * q, lo + 3 * q
            c1, c2, c3 = count(t1), count(t2), count(t3)
            new_lo = jnp.where(c3 >= k, t3,
                               jnp.where(c2 >= k, t2,
                                         jnp.where(c1 >= k, t1, lo)))
            new_hi = jnp.where(c1 < k, t1 - 1,
                               jnp.where(c2 < k, t2 - 1,
                                         jnp.where(c3 < k, t3 - 1, hi)))
            return new_lo, new_hi

        lo, _ = jax.lax.while_loop(lambda c: c[0] < c[1], body, (lo0, hi0))

        sel = bits >= lo
        v = jax.lax.bitcast_convert_type(bits, jnp.float32)
        total = jnp.sum(jnp.sum(jnp.where(sel, v, 0.0), axis=0))
        cnt = jnp.sum(jnp.sum(sel.astype(jnp.int32), axis=0))
        out_ref[...] = jnp.reshape(total / (cnt.astype(jnp.float32) + _EPS),
                                   (1, 1))


def kernel(pred, target):
    B, C, H, W = pred.shape
    TH = min(128, H)
    N = B * H * W
    k = int(N * _OHEM_RATIO)
    n_h = H // TH
    n_steps = B * n_h
    out = pl.pallas_call(
        functools.partial(_fused_kernel, k=k, n_steps=n_steps),
        grid=(B, n_h),
        in_specs=[
            pl.BlockSpec((1, C, TH, W), lambda b, h: (b, 0, h, 0)),
            pl.BlockSpec((1, TH, W), lambda b, h: (b, h, 0)),
        ],
        out_specs=pl.BlockSpec((1, 1), lambda b, h: (0, 0)),
        out_shape=jax.ShapeDtypeStruct((1, 1), jnp.float32),
        scratch_shapes=[pltpu.VMEM((n_steps, TH, W), jnp.int32),
                        pltpu.SMEM((3,), jnp.int32),
                        pltpu.VMEM((_NR, TH, W), jnp.int32)],
    )(pred, target)
    return out[0, 0]
